# baseline (speedup 1.0000x reference)
import jax
import jax.numpy as jnp
from jax.experimental import pallas as pl
from jax.experimental.pallas import tpu as pltpu
from jax.experimental.pallas import tpu_sc as plsc

_W = 256


def _sc_add(x2d, pos_table, S):
    R, D = x2d.shape
    BR = 16

    mesh = plsc.VectorSubcoreMesh(core_axis_name="core",
                                  subcore_axis_name="subcore")

    @pl.kernel(out_type=jax.ShapeDtypeStruct((R, D), x2d.dtype), mesh=mesh)
    def run(x_hbm, pe_hbm, o_hbm):
        def body(x_vmem, pe_vmem, o_vmem):
            for r in range(BR):
                for c in range(0, D, _W):
                    slc = (pl.ds(r, 1), pl.ds(c, _W))
                    o_vmem.at[*slc][...] = (
                        x_vmem.at[*slc][...] + pe_vmem.at[*slc][...]
                    )

        nsb = S // BR
        pltpu.emit_pipeline(
            body,
            grid=(nsb, R // S),
            in_specs=[
                pl.BlockSpec((BR, D), lambda j, b: (b * nsb + j, 0)),
                pl.BlockSpec((BR, D), lambda j, b: (j, 0)),
            ],
            out_specs=[pl.BlockSpec((BR, D), lambda j, b: (b * nsb + j, 0))],
            core_axis_name=("core", "subcore"),
            dimension_semantics=(pltpu.PARALLEL, pltpu.ARBITRARY),
        )(x_hbm, pe_hbm, o_hbm)

    return run(x2d, pos_table)


def kernel(x, pos_table):
    B, S, D = x.shape
    x2d = x.reshape(B * S, D)
    out = _sc_add(x2d, pos_table, S)
    return out.reshape(B, S, D)


# speedup vs baseline: 2.0298x; 2.0298x over previous
import jax
import jax.numpy as jnp
from jax.experimental import pallas as pl
from jax.experimental.pallas import tpu as pltpu
from jax.experimental.pallas import tpu_sc as plsc

_W = 256
_BR = 16


def _sc_add(x2d, pos_table, S):
    R, D = x2d.shape
    BR = _BR

    mesh = plsc.VectorSubcoreMesh(core_axis_name="core",
                                  subcore_axis_name="subcore")

    @pl.kernel(out_type=jax.ShapeDtypeStruct((R, D), x2d.dtype), mesh=mesh)
    def run(x_hbm, pe_hbm, o_hbm):
        def body(x_vmem, pe_vmem, o_vmem):
            @pl.loop(0, BR)
            def _(r):
                @pl.loop(0, D, step=_W)
                def _(c):
                    slc = (pl.ds(r, 1), pl.ds(c, _W))
                    o_vmem.at[*slc][...] = (
                        x_vmem.at[*slc][...] + pe_vmem.at[*slc][...]
                    )

        nsb = S // BR
        pltpu.emit_pipeline(
            body,
            grid=(nsb, R // S),
            in_specs=[
                pl.BlockSpec((BR, D), lambda j, b: (b * nsb + j, 0)),
                pl.BlockSpec((BR, D), lambda j, b: (j, 0)),
            ],
            out_specs=[pl.BlockSpec((BR, D), lambda j, b: (b * nsb + j, 0))],
            core_axis_name=("core", "subcore"),
            dimension_semantics=(pltpu.PARALLEL, pltpu.ARBITRARY),
        )(x_hbm, pe_hbm, o_hbm)

    return run(x2d, pos_table)


def kernel(x, pos_table):
    B, S, D = x.shape
    x2d = x.reshape(B * S, D)
    out = _sc_add(x2d, pos_table, S)
    return out.reshape(B, S, D)
